# baseline (speedup 1.0000x reference)
import jax
import jax.numpy as jnp
from jax.experimental import pallas as pl

_NUM_PARAMS = 128
_STEPS = 32
_BATCH = 16384
_BB = 256


def _onehot_kernel(x_ref, o_ref):
    x = x_ref[...]
    idx = jnp.floor(x * jnp.float32(_STEPS))
    for c in range(_STEPS):
        o_ref[:, c, :] = jnp.where(idx == jnp.float32(c), jnp.float32(1.0), jnp.float32(0.0))


def kernel(x):
    assert x.shape == (_BATCH, _NUM_PARAMS) and x.dtype == jnp.float32
    grid = (_BATCH // _BB,)
    return pl.pallas_call(
        _onehot_kernel,
        grid=grid,
        in_specs=[pl.BlockSpec((_BB, _NUM_PARAMS), lambda i: (i, 0))],
        out_specs=pl.BlockSpec((_BB, _STEPS, _NUM_PARAMS), lambda i: (i, 0, 0)),
        out_shape=jax.ShapeDtypeStruct((_BATCH, _STEPS, _NUM_PARAMS), jnp.float32),
    )(x)


# speedup vs baseline: 3.2800x; 3.2800x over previous
import jax
import jax.numpy as jnp
from jax.experimental import pallas as pl

_NUM_PARAMS = 128
_STEPS = 32
_BATCH = 16384
_BB = 256


def _onehot_kernel(x_ref, o_ref):
    x = x_ref[...]
    idx = jnp.floor(x * jnp.float32(_STEPS)).astype(jnp.int32)
    c = jax.lax.broadcasted_iota(jnp.int32, (_BB, _STEPS, _NUM_PARAMS), 1)
    o_ref[...] = jnp.where(idx[:, None, :] == c, jnp.float32(1.0), jnp.float32(0.0))


def kernel(x):
    assert x.shape == (_BATCH, _NUM_PARAMS) and x.dtype == jnp.float32
    grid = (_BATCH // _BB,)
    return pl.pallas_call(
        _onehot_kernel,
        grid=grid,
        in_specs=[pl.BlockSpec((_BB, _NUM_PARAMS), lambda i: (i, 0))],
        out_specs=pl.BlockSpec((_BB, _STEPS, _NUM_PARAMS), lambda i: (i, 0, 0)),
        out_shape=jax.ShapeDtypeStruct((_BATCH, _STEPS, _NUM_PARAMS), jnp.float32),
    )(x)


# speedup vs baseline: 3.5222x; 1.0738x over previous
import jax
import jax.numpy as jnp
from jax.experimental import pallas as pl

_NUM_PARAMS = 128
_STEPS = 32
_BATCH = 16384
_BB = 512


def _onehot_kernel(x_ref, o_ref):
    x = x_ref[...]
    idx = jnp.floor(x * jnp.float32(_STEPS)).astype(jnp.int32)
    c = jax.lax.broadcasted_iota(jnp.int32, (_BB, _STEPS, _NUM_PARAMS), 1)
    o_ref[...] = jnp.where(idx[:, None, :] == c, jnp.float32(1.0), jnp.float32(0.0))


def kernel(x):
    assert x.shape == (_BATCH, _NUM_PARAMS) and x.dtype == jnp.float32
    grid = (_BATCH // _BB,)
    return pl.pallas_call(
        _onehot_kernel,
        grid=grid,
        in_specs=[pl.BlockSpec((_BB, _NUM_PARAMS), lambda i: (i, 0))],
        out_specs=pl.BlockSpec((_BB, _STEPS, _NUM_PARAMS), lambda i: (i, 0, 0)),
        out_shape=jax.ShapeDtypeStruct((_BATCH, _STEPS, _NUM_PARAMS), jnp.float32),
    )(x)
